# Initial kernel scaffold; baseline (speedup 1.0000x reference)
import jax
import jax.numpy as jnp
from jax.experimental import pallas as pl


def kernel(x, edge_index, batch, W1, a_src1, a_dst1, b1, W2, a_src2, a_dst2, b2, Wl, bl):
    raise NotImplementedError("write your pallas kernel here")


# speedup vs baseline: 14.1307x; 14.1307x over previous
import jax
import jax.numpy as jnp
from jax import lax
from jax.experimental import pallas as pl
from jax.experimental.pallas import tpu as pltpu
from jax.experimental.pallas import tpu_sc as plsc

N = 10000
D = 128
H = 64
HH = H // 2
NCLS = 10
NG = 64
E = 320000

NPAD = 10112
N2 = 2 * NPAD
NW = 16
EW = 20736
EPAD = NW * EW
SLAB = 2592
NSLAB = EW // SLAB
CH = 64
TOP = EW + CH
TOPC = TOP // CH
BUF = TOP + 16
RPT = NPAD // 16
CPT = N2 // 16
F32 = jnp.float32


def _attn_tables(h, asrc_c, adst_c, gt_ref, scal_ref):
    a_s = h @ asrc_c
    a_d = h @ adst_c
    ms = jnp.max(a_s)
    t_ub = ms + jnp.max(a_d)
    m = jnp.where(t_ub > 0.0, t_ub, 0.2 * t_ub)
    u = jnp.exp(a_s - ms)
    u2 = jnp.exp(0.2 * (a_s - ms))
    v = jnp.exp(a_d - (m - ms))
    v2 = jnp.exp(0.2 * a_d - (m - 0.2 * ms))
    gt_ref[...] = jnp.zeros((2, NPAD, H), F32)
    gt_ref[0, 0:N, :] = h * u
    gt_ref[1, 0:N, :] = h * u2
    scal_ref[...] = jnp.zeros((NPAD + 16, 8), F32)
    scal_ref[0:N, 0:1] = a_s
    scal_ref[0:N, 1:2] = a_d
    scal_ref[0:N, 2:3] = v
    scal_ref[0:N, 3:4] = v2
    scal_ref[NPAD:NPAD + 16, 0:1] = jnp.full((16, 1), ms, F32)


def _prep_body(x_ref, w_ref, asrc_ref, adst_ref, gt_ref, scal_ref):
    h = x_ref[...] @ w_ref[...]
    _attn_tables(h, asrc_ref[...], adst_ref[...], gt_ref, scal_ref)


def _gat_out(p_ref, c_ref, scal_ref, b_ref):
    p1 = p_ref[0:N, 0:H]
    p2 = p_ref[0:N, H:2 * H]
    v = scal_ref[0:N, 2:3]
    v2 = scal_ref[0:N, 3:4]
    num = v * p1 + v2 * p2
    den = v * c_ref[0:N, 0:1] + v2 * c_ref[0:N, 1:2]
    return jnp.maximum(num / den + b_ref[...], 0.0)


def _mid_body(p_ref, c_ref, scal_in_ref, b_ref, w_ref, asrc_ref, adst_ref,
              gt_ref, scal_ref):
    x2 = _gat_out(p_ref, c_ref, scal_in_ref, b_ref)
    h = x2 @ w_ref[...]
    _attn_tables(h, asrc_ref[...], adst_ref[...], gt_ref, scal_ref)


def _final_body(p_ref, c_ref, scal_ref, b_ref, batch_ref, wl_ref, bl_ref,
                o_ref):
    x3 = _gat_out(p_ref, c_ref, scal_ref, b_ref)
    onehot = (batch_ref[...] ==
              lax.broadcasted_iota(jnp.int32, (N, NG), 1)).astype(F32)
    s = lax.dot_general(onehot, x3, (((0,), (0,)), ((), ())))
    cnt = lax.dot_general(onehot, jnp.ones((N, 1), F32),
                          (((0,), (0,)), ((), ())))
    pooled = s / jnp.maximum(cnt, 1.0)
    o_ref[...] = pooled @ wl_ref[...] + bl_ref[...]


_sds = jax.ShapeDtypeStruct
_TAB_OUT = [_sds((2, NPAD, H), F32), _sds((NPAD + 16, 8), F32)]

_prep_call = pl.pallas_call(_prep_body, out_shape=_TAB_OUT)
_mid_call = pl.pallas_call(_mid_body, out_shape=_TAB_OUT)
_final_call = pl.pallas_call(_final_body, out_shape=_sds((NG, NCLS), F32))


def _edge_body(src_hbm, dst_hbm, as_hbm, ad_hbm, zp_hbm, glo_hbm, ghi_hbm,
               p1l_out, p1h_out, p2l_out, p2h_out, c_out,
               src_v, dst_v, as_v, ad_v, c_v, pg_v, sg_v, rows_v, idn_v,
               acc_sp, csp, sem):
    sid = lax.axis_index("s")
    pltpu.sync_copy(as_hbm, as_v)
    pltpu.sync_copy(ad_hbm, ad_v)

    rbase = sid * RPT
    cbase = sid * CPT
    pltpu.sync_copy(zp_hbm, acc_sp.at[pl.ds(rbase, RPT)])

    zeros16 = jnp.zeros((16,), F32)

    def zc(i, carry):
        c_v[pl.ds(i * 16, 16)] = zeros16
        return carry

    lax.fori_loop(0, N2 // 16, zc, 0)
    pltpu.sync_copy(c_v.at[pl.ds(cbase, CPT)], csp.at[pl.ds(cbase, CPT)])

    plsc.subcore_barrier()

    msv = as_v[pl.ds(NPAD, 16)]
    zero_i = jnp.zeros((16,), jnp.int32)
    npad_i = jnp.full((16,), NPAD, jnp.int32)
    one_f = jnp.full((16,), 1.0, F32)
    fifth_f = jnp.full((16,), 0.2, F32)
    ebase = sid * EW

    def slab_body(sl, cpcn):
        pltpu.sync_copy(src_hbm.at[pl.ds(ebase + sl * SLAB, SLAB)], src_v)
        pltpu.sync_copy(dst_hbm.at[pl.ds(ebase + sl * SLAB, SLAB)], dst_v)

        def egroup(g, cpcn):
            cp, cn = cpcn
            s = src_v[pl.ds(g * 16, 16)]
            d = dst_v[pl.ds(g * 16, 16)]
            a1 = plsc.load_gather(as_v, [s])
            a2 = plsc.load_gather(ad_v, [d])
            pos = (a1 + a2) > 0.0
            off = jnp.where(pos, zero_i, npad_i)
            w = jnp.where(pos, one_f, fifth_f)
            val = jnp.exp(w * (a1 - msv))
            plsc.addupdate_scatter(c_v, [d + off], val)
            gi = s + off
            pc = plsc.all_reduce_population_count(pos)
            kp = lax.squeeze(lax.slice(pc, (0,), (1,)), (0,))
            kn = 16 - kp
            plsc.store_compressed(pg_v.at[pl.ds(cp, 16)], gi, mask=pos)
            plsc.store_compressed(sg_v.at[pl.ds(cp, 16)], d, mask=pos)
            neg = jnp.logical_not(pos)
            plsc.store_compressed(pg_v.at[pl.ds(cn - kn, 16)], gi, mask=neg)
            plsc.store_compressed(sg_v.at[pl.ds(cn - kn, 16)], d, mask=neg)
            return (cp + kp, cn - kn)

        return lax.fori_loop(0, SLAB // 16, egroup, cpcn)

    cp, cn = lax.fori_loop(0, NSLAB, slab_body, (0, TOP))

    dummy_p = jnp.full((16,), N, jnp.int32)
    dummy_n = jnp.full((16,), NPAD + N, jnp.int32)

    def padp(i, carry):
        pg_v[pl.ds(cp + i * 16, 16)] = dummy_p
        sg_v[pl.ds(cp + i * 16, 16)] = dummy_p
        return carry

    def padn(i, carry):
        pg_v[pl.ds(cn - CH + i * 16, 16)] = dummy_n
        sg_v[pl.ds(cn - CH + i * 16, 16)] = dummy_p
        return carry

    lax.fori_loop(0, CH // 16, padp, 0)
    lax.fori_loop(0, CH // 16, padn, 0)

    sixteen = jnp.full((16,), 16, jnp.int32)

    def bld(k, vec):
        idn_v[pl.ds(k * 16, 16)] = vec
        return vec + sixteen

    lax.fori_loop(0, N2 // 16, bld, lax.iota(jnp.int32, 16))

    def cchunk(j, carry):
        pltpu.sync_copy(c_v.at[pl.ds(j * 128, 128)],
                        csp.at[idn_v.at[pl.ds(j * 128, 128)]], add=True)
        return carry

    lax.fori_loop(0, N2 // 128, cchunk, 0)

    npc = (cp + CH - 1) // CH
    nfs = cn // CH

    def make_rchunk(tab):
        def rchunk(j, carry):
            idx = pg_v.at[pl.ds(j * CH, CH)]
            pltpu.async_copy(tab.at[idx], rows_v, sem).wait()
            pltpu.sync_copy(rows_v, acc_sp.at[sg_v.at[pl.ds(j * CH, CH)]],
                            add=True)
            return carry

        return rchunk

    def sweep_done(p_out):
        plsc.subcore_barrier()
        pltpu.sync_copy(acc_sp.at[pl.ds(rbase, RPT)],
                        p_out.at[pl.ds(rbase, RPT)])
        pltpu.sync_copy(zp_hbm, acc_sp.at[pl.ds(rbase, RPT)])
        plsc.subcore_barrier()

    lax.fori_loop(0, npc, make_rchunk(glo_hbm), 0)
    sweep_done(p1l_out)
    lax.fori_loop(0, npc, make_rchunk(ghi_hbm), 0)
    sweep_done(p1h_out)
    lax.fori_loop(nfs, TOPC, make_rchunk(glo_hbm), 0)
    sweep_done(p2l_out)
    lax.fori_loop(nfs, TOPC, make_rchunk(ghi_hbm), 0)
    plsc.subcore_barrier()
    pltpu.sync_copy(acc_sp.at[pl.ds(rbase, RPT)], p2h_out.at[pl.ds(rbase, RPT)])
    pltpu.sync_copy(csp.at[pl.ds(cbase, CPT)], c_out.at[pl.ds(cbase, CPT)])


_edge_call = pl.kernel(
    _edge_body,
    out_type=[_sds((NPAD, HH), F32), _sds((NPAD, HH), F32),
              _sds((NPAD, HH), F32), _sds((NPAD, HH), F32),
              _sds((N2,), F32)],
    mesh=plsc.VectorSubcoreMesh(core_axis_name="c", subcore_axis_name="s",
                                num_cores=1),
    compiler_params=pltpu.CompilerParams(needs_layout_passes=False,
                                         use_tc_tiling_on_sc=False),
    scratch_types=[
        pltpu.VMEM((SLAB,), jnp.int32),
        pltpu.VMEM((SLAB,), jnp.int32),
        pltpu.VMEM((NPAD + 16,), F32),
        pltpu.VMEM((NPAD,), F32),
        pltpu.VMEM((N2,), F32),
        pltpu.VMEM((BUF,), jnp.int32),
        pltpu.VMEM((BUF,), jnp.int32),
        pltpu.VMEM((CH, HH), F32),
        pltpu.VMEM((N2,), jnp.int32),
        pltpu.VMEM_SHARED((NPAD, HH), F32),
        pltpu.VMEM_SHARED((N2,), F32),
        pltpu.SemaphoreType.DMA,
    ],
)


@jax.jit
def kernel(x, edge_index, batch, W1, a_src1, a_dst1, b1,
           W2, a_src2, a_dst2, b2, Wl, bl):
    loop = jnp.arange(N, dtype=jnp.int32)
    padv = jnp.full((EPAD - E - N,), N, jnp.int32)
    src = jnp.concatenate([edge_index[0].astype(jnp.int32), loop, padv])
    dst = jnp.concatenate([edge_index[1].astype(jnp.int32), loop, padv])
    zp = jnp.zeros((RPT, HH), F32)

    def edge_phase(gt, scal):
        glo = gt[:, :, 0:HH].reshape(N2, HH)
        ghi = gt[:, :, HH:H].reshape(N2, HH)
        as_flat = scal[:, 0]
        ad_flat = scal[0:NPAD, 1]
        p1l, p1h, p2l, p2h, c_ = _edge_call(src, dst, as_flat, ad_flat, zp,
                                            glo, ghi)
        p = jnp.concatenate([p1l, p1h, p2l, p2h], axis=1)
        c2 = jnp.stack([c_[0:NPAD], c_[NPAD:N2]], axis=1)
        return p, c2

    gt1, scal1 = _prep_call(x, W1, a_src1.reshape(H, 1), a_dst1.reshape(H, 1))
    p1, c1 = edge_phase(gt1, scal1)
    gt2, scal2 = _mid_call(p1, c1, scal1, b1, W2, a_src2.reshape(H, 1),
                           a_dst2.reshape(H, 1))
    p2, c2 = edge_phase(gt2, scal2)
    return _final_call(p2, c2, scal2, b2,
                       batch.reshape(N, 1).astype(jnp.int32), Wl, bl)


# speedup vs baseline: 21.6593x; 1.5328x over previous
import jax
import jax.numpy as jnp
from jax import lax
from jax.experimental import pallas as pl
from jax.experimental.pallas import tpu as pltpu
from jax.experimental.pallas import tpu_sc as plsc

N = 10000
D = 128
H = 64
HH = H // 2
NCLS = 10
NG = 64
E = 320000

NPAD = 10112
N2 = 2 * NPAD
NW = 16
EW = 20736
EPAD = NW * EW
SLAB = 2592
NSLAB = EW // SLAB
CH = 64
TOP = EW + CH
TOPC = TOP // CH
BUF = TOP + 16
RPT = NPAD // 16
CPT = N2 // 16
F32 = jnp.float32


def _attn_tables(h, asrc_c, adst_c, gt_ref, scal_ref):
    a_s = h @ asrc_c
    a_d = h @ adst_c
    ms = jnp.max(a_s)
    t_ub = ms + jnp.max(a_d)
    m = jnp.where(t_ub > 0.0, t_ub, 0.2 * t_ub)
    u = jnp.exp(a_s - ms)
    u2 = jnp.exp(0.2 * (a_s - ms))
    v = jnp.exp(a_d - (m - ms))
    v2 = jnp.exp(0.2 * a_d - (m - 0.2 * ms))
    gt_ref[...] = jnp.zeros((2, NPAD, H), F32)
    gt_ref[0, 0:N, :] = h * u
    gt_ref[1, 0:N, :] = h * u2
    scal_ref[...] = jnp.zeros((NPAD + 16, 8), F32)
    scal_ref[0:N, 0:1] = a_s
    scal_ref[0:N, 1:2] = a_d
    scal_ref[0:N, 2:3] = v
    scal_ref[0:N, 3:4] = v2
    scal_ref[NPAD:NPAD + 16, 0:1] = jnp.full((16, 1), ms, F32)


def _prep_body(x_ref, w_ref, asrc_ref, adst_ref, gt_ref, scal_ref):
    h = x_ref[...] @ w_ref[...]
    _attn_tables(h, asrc_ref[...], adst_ref[...], gt_ref, scal_ref)


def _gat_out(p_ref, c_ref, scal_ref, b_ref):
    p1 = p_ref[0:N, 0:H]
    p2 = p_ref[0:N, H:2 * H]
    v = scal_ref[0:N, 2:3]
    v2 = scal_ref[0:N, 3:4]
    num = v * p1 + v2 * p2
    den = v * c_ref[0:N, 0:1] + v2 * c_ref[0:N, 1:2]
    return jnp.maximum(num / den + b_ref[...], 0.0)


def _mid_body(p_ref, c_ref, scal_in_ref, b_ref, w_ref, asrc_ref, adst_ref,
              gt_ref, scal_ref):
    x2 = _gat_out(p_ref, c_ref, scal_in_ref, b_ref)
    h = x2 @ w_ref[...]
    _attn_tables(h, asrc_ref[...], adst_ref[...], gt_ref, scal_ref)


def _final_body(p_ref, c_ref, scal_ref, b_ref, batch_ref, wl_ref, bl_ref,
                o_ref):
    x3 = _gat_out(p_ref, c_ref, scal_ref, b_ref)
    onehot = (batch_ref[...] ==
              lax.broadcasted_iota(jnp.int32, (N, NG), 1)).astype(F32)
    s = lax.dot_general(onehot, x3, (((0,), (0,)), ((), ())))
    cnt = lax.dot_general(onehot, jnp.ones((N, 1), F32),
                          (((0,), (0,)), ((), ())))
    pooled = s / jnp.maximum(cnt, 1.0)
    o_ref[...] = pooled @ wl_ref[...] + bl_ref[...]


_sds = jax.ShapeDtypeStruct
_TAB_OUT = [_sds((2, NPAD, H), F32), _sds((NPAD + 16, 8), F32)]

_prep_call = pl.pallas_call(_prep_body, out_shape=_TAB_OUT)
_mid_call = pl.pallas_call(_mid_body, out_shape=_TAB_OUT)
_final_call = pl.pallas_call(_final_body, out_shape=_sds((NG, NCLS), F32))


def _edge_body(src_hbm, dst_hbm, as_hbm, ad_hbm, zp_hbm, gt_hbm,
               p_out, c_out,
               src_v, dst_v, as_v, ad_v, c2d_v, pg_v, sg_v, rows_v, idn_v,
               acc_sp, gs0, gs1):
    sid = lax.axis_index("s")
    pltpu.sync_copy(as_hbm, as_v)
    pltpu.sync_copy(ad_hbm, ad_v)

    rbase = sid * RPT
    cbase = sid * CPT
    pltpu.sync_copy(zp_hbm, acc_sp.at[pl.ds(rbase, RPT)])

    zeros16 = jnp.zeros((16,), F32)

    def zc(i, carry):
        c2d_v[i // 2, pl.ds((i % 2) * 16, 16)] = zeros16
        return carry

    lax.fori_loop(0, 1280, zc, 0)

    plsc.subcore_barrier()

    msv = as_v[pl.ds(NPAD, 16)]
    zero_i = jnp.zeros((16,), jnp.int32)
    npad_i = jnp.full((16,), NPAD, jnp.int32)
    one_f = jnp.full((16,), 1.0, F32)
    fifth_f = jnp.full((16,), 0.2, F32)
    ebase = sid * EW

    def slab_body(sl, cpcn):
        pltpu.sync_copy(src_hbm.at[pl.ds(ebase + sl * SLAB, SLAB)], src_v)
        pltpu.sync_copy(dst_hbm.at[pl.ds(ebase + sl * SLAB, SLAB)], dst_v)

        def egroup(g, cpcn):
            cp, cn = cpcn
            s = src_v[pl.ds(g * 16, 16)]
            d = dst_v[pl.ds(g * 16, 16)]
            a1 = plsc.load_gather(as_v, [s])
            a2 = plsc.load_gather(ad_v, [d])
            pos = (a1 + a2) > 0.0
            off = jnp.where(pos, zero_i, npad_i)
            w = jnp.where(pos, one_f, fifth_f)
            val = jnp.exp(w * (a1 - msv))
            ci = d + off
            plsc.addupdate_scatter(
                c2d_v, [lax.shift_right_logical(ci, 5),
                        lax.bitwise_and(ci, jnp.full((16,), 31, jnp.int32))],
                val)
            gi = s + off
            pc = plsc.all_reduce_population_count(pos)
            kp = lax.squeeze(lax.slice(pc, (0,), (1,)), (0,))
            kn = 16 - kp
            plsc.store_compressed(pg_v.at[pl.ds(cp, 16)], gi, mask=pos)
            plsc.store_compressed(sg_v.at[pl.ds(cp, 16)], d, mask=pos)
            neg = jnp.logical_not(pos)
            plsc.store_compressed(pg_v.at[pl.ds(cn - kn, 16)], gi, mask=neg)
            plsc.store_compressed(sg_v.at[pl.ds(cn - kn, 16)], d, mask=neg)
            return (cp + kp, cn - kn)

        return lax.fori_loop(0, SLAB // 16, egroup, cpcn)

    cp, cn = lax.fori_loop(0, NSLAB, slab_body, (0, TOP))

    dummy_p = jnp.full((16,), N, jnp.int32)
    dummy_n = jnp.full((16,), NPAD + N, jnp.int32)

    def padp(i, carry):
        pg_v[pl.ds(cp + i * 16, 16)] = dummy_p
        sg_v[pl.ds(cp + i * 16, 16)] = dummy_p
        return carry

    def padn(i, carry):
        pg_v[pl.ds(cn - CH + i * 16, 16)] = dummy_n
        sg_v[pl.ds(cn - CH + i * 16, 16)] = dummy_p
        return carry

    lax.fori_loop(0, CH // 16, padp, 0)
    lax.fori_loop(0, CH // 16, padn, 0)

    sixteen = jnp.full((16,), 16, jnp.int32)

    def bld(k, vec):
        idn_v[pl.ds(k * 16, 16)] = vec
        return vec + sixteen

    lax.fori_loop(0, 40, bld, lax.iota(jnp.int32, 16))

    npc = (cp + CH - 1) // CH
    nfs = cn // CH

    def sweep_body(s, carry):
        lo = jnp.where(s < 2, 0, nfs)
        hi = jnp.where(s < 2, npc, TOPC)
        tab = gt_hbm.at[s % 2]

        def start_g(j, buf, gsem):
            pltpu.async_copy(tab.at[pg_v.at[pl.ds(j * CH, CH)]], buf, gsem)

        def wait_g(buf, gsem):
            pltpu.make_async_copy(tab.at[pg_v.at[pl.ds(0, CH)]], buf,
                                  gsem).wait()

        def scat(j, buf):
            pltpu.sync_copy(buf, acc_sp.at[sg_v.at[pl.ds(j * CH, CH)]],
                            add=True)

        @pl.when(hi > lo)
        def _():
            start_g(lo, rows_v.at[0], gs0)

        def body(k, carry2):
            j0 = lo + 2 * k
            j1 = j0 + 1

            @pl.when(j1 < hi)
            def _():
                start_g(j1, rows_v.at[1], gs1)

            wait_g(rows_v.at[0], gs0)
            scat(j0, rows_v.at[0])

            @pl.when(j1 < hi)
            def _():
                @pl.when(j1 + 1 < hi)
                def __():
                    start_g(j1 + 1, rows_v.at[0], gs0)

                wait_g(rows_v.at[1], gs1)
                scat(j1, rows_v.at[1])

            return carry2

        lax.fori_loop(0, (hi - lo + 1) // 2, body, 0)

        plsc.subcore_barrier()
        pltpu.sync_copy(acc_sp.at[pl.ds(rbase, RPT)],
                        p_out.at[s, pl.ds(rbase, RPT)])
        pltpu.sync_copy(zp_hbm, acc_sp.at[pl.ds(rbase, RPT)])
        plsc.subcore_barrier()
        return carry

    lax.fori_loop(0, 4, sweep_body, 0)

    def cchunk(j, carry):
        pltpu.sync_copy(c2d_v.at[pl.ds(j * 128, 128)],
                        acc_sp.at[idn_v.at[pl.ds(j * 128, 128)]], add=True)
        return carry

    lax.fori_loop(0, 5, cchunk, 0)
    plsc.subcore_barrier()

    @pl.when(sid < 8)
    def _():
        pltpu.sync_copy(acc_sp.at[pl.ds(sid * 79, 79)],
                        c_out.at[pl.ds(sid * 79, 79)])


_edge_call = pl.kernel(
    _edge_body,
    out_type=[_sds((4, NPAD, HH), F32), _sds((632, 32), F32)],
    mesh=plsc.VectorSubcoreMesh(core_axis_name="c", subcore_axis_name="s",
                                num_cores=1),
    compiler_params=pltpu.CompilerParams(needs_layout_passes=False,
                                         use_tc_tiling_on_sc=False),
    scratch_types=[
        pltpu.VMEM((SLAB,), jnp.int32),
        pltpu.VMEM((SLAB,), jnp.int32),
        pltpu.VMEM((NPAD + 16,), F32),
        pltpu.VMEM((NPAD,), F32),
        pltpu.VMEM((640, 32), F32),
        pltpu.VMEM((BUF,), jnp.int32),
        pltpu.VMEM((BUF,), jnp.int32),
        pltpu.VMEM((2, CH, HH), F32),
        pltpu.VMEM((640,), jnp.int32),
        pltpu.VMEM_SHARED((NPAD, HH), F32),
        pltpu.SemaphoreType.DMA,
        pltpu.SemaphoreType.DMA,
    ],
)


@jax.jit
def kernel(x, edge_index, batch, W1, a_src1, a_dst1, b1,
           W2, a_src2, a_dst2, b2, Wl, bl):
    loop = jnp.arange(N, dtype=jnp.int32)
    padv = jnp.full((EPAD - E - N,), N, jnp.int32)
    src = jnp.concatenate([edge_index[0].astype(jnp.int32), loop, padv])
    dst = jnp.concatenate([edge_index[1].astype(jnp.int32), loop, padv])
    zp = jnp.zeros((RPT, HH), F32)

    def edge_phase(gt, scal):
        glo = gt[:, :, 0:HH].reshape(N2, HH)
        ghi = gt[:, :, HH:H].reshape(N2, HH)
        gt_all = jnp.stack([glo, ghi])
        as_flat = scal[:, 0]
        ad_flat = scal[0:NPAD, 1]
        p4, c_ = _edge_call(src, dst, as_flat, ad_flat, zp, gt_all)
        c_ = c_.reshape(N2)
        p = jnp.concatenate([p4[0], p4[1], p4[2], p4[3]], axis=1)
        c2 = jnp.stack([c_[0:NPAD], c_[NPAD:N2]], axis=1)
        return p, c2

    gt1, scal1 = _prep_call(x, W1, a_src1.reshape(H, 1), a_dst1.reshape(H, 1))
    p1, c1 = edge_phase(gt1, scal1)
    gt2, scal2 = _mid_call(p1, c1, scal1, b1, W2, a_src2.reshape(H, 1),
                           a_dst2.reshape(H, 1))
    p2, c2 = edge_phase(gt2, scal2)
    return _final_call(p2, c2, scal2, b2,
                       batch.reshape(N, 1).astype(jnp.int32), Wl, bl)


# speedup vs baseline: 25.4029x; 1.1728x over previous
import jax
import jax.numpy as jnp
from jax import lax
from jax.experimental import pallas as pl
from jax.experimental.pallas import tpu as pltpu
from jax.experimental.pallas import tpu_sc as plsc

N = 10000
D = 128
H = 64
HH = H // 2
NCLS = 10
NG = 64
E = 320000

NPAD = 10112
N2 = 2 * NPAD
NW = 16
EW = 20736
EPAD = NW * EW
SLAB = 2592
NSLAB = EW // SLAB
CH = 128
TOP = EW + CH
TOPC = TOP // CH
BUF = TOP + 16
RPT = NPAD // 16
CPT = N2 // 16
F32 = jnp.float32


def _attn_tables(h, asrc_c, adst_c, gt_ref, scal_ref):
    a_s = h @ asrc_c
    a_d = h @ adst_c
    ms = jnp.max(a_s)
    t_ub = ms + jnp.max(a_d)
    m = jnp.where(t_ub > 0.0, t_ub, 0.2 * t_ub)
    u = jnp.exp(a_s - ms)
    u2 = jnp.exp(0.2 * (a_s - ms))
    v = jnp.exp(a_d - (m - ms))
    v2 = jnp.exp(0.2 * a_d - (m - 0.2 * ms))
    gt_ref[...] = jnp.zeros((2, NPAD, H), F32)
    gt_ref[0, 0:N, :] = h * u
    gt_ref[1, 0:N, :] = h * u2
    scal_ref[...] = jnp.zeros((NPAD + 16, 8), F32)
    scal_ref[0:N, 0:1] = a_s
    scal_ref[0:N, 1:2] = a_d
    scal_ref[0:N, 2:3] = v
    scal_ref[0:N, 3:4] = v2
    scal_ref[NPAD:NPAD + 16, 0:1] = jnp.full((16, 1), ms, F32)


def _prep_body(x_ref, w_ref, asrc_ref, adst_ref, gt_ref, scal_ref):
    h = x_ref[...] @ w_ref[...]
    _attn_tables(h, asrc_ref[...], adst_ref[...], gt_ref, scal_ref)


def _gat_out(p_ref, c_ref, scal_ref, b_ref):
    p1 = p_ref[0:N, 0:H]
    p2 = p_ref[0:N, H:2 * H]
    v = scal_ref[0:N, 2:3]
    v2 = scal_ref[0:N, 3:4]
    num = v * p1 + v2 * p2
    den = v * c_ref[0:N, 0:1] + v2 * c_ref[0:N, 1:2]
    return jnp.maximum(num / den + b_ref[...], 0.0)


def _mid_body(p_ref, c_ref, scal_in_ref, b_ref, w_ref, asrc_ref, adst_ref,
              gt_ref, scal_ref):
    x2 = _gat_out(p_ref, c_ref, scal_in_ref, b_ref)
    h = x2 @ w_ref[...]
    _attn_tables(h, asrc_ref[...], adst_ref[...], gt_ref, scal_ref)


def _final_body(p_ref, c_ref, scal_ref, b_ref, batch_ref, wl_ref, bl_ref,
                o_ref):
    x3 = _gat_out(p_ref, c_ref, scal_ref, b_ref)
    onehot = (batch_ref[...] ==
              lax.broadcasted_iota(jnp.int32, (N, NG), 1)).astype(F32)
    s = lax.dot_general(onehot, x3, (((0,), (0,)), ((), ())))
    cnt = lax.dot_general(onehot, jnp.ones((N, 1), F32),
                          (((0,), (0,)), ((), ())))
    pooled = s / jnp.maximum(cnt, 1.0)
    o_ref[...] = pooled @ wl_ref[...] + bl_ref[...]


_sds = jax.ShapeDtypeStruct
_TAB_OUT = [_sds((2, NPAD, H), F32), _sds((NPAD + 16, 8), F32)]

_prep_call = pl.pallas_call(_prep_body, out_shape=_TAB_OUT)
_mid_call = pl.pallas_call(_mid_body, out_shape=_TAB_OUT)
_final_call = pl.pallas_call(_final_body, out_shape=_sds((NG, NCLS), F32))


def _edge_body(src_hbm, dst_hbm, as_hbm, ad_hbm, zp_hbm, gt_hbm,
               p_out, c_out,
               src_v, dst_v, as_v, ad_v, c2d_v, pg_v, sg_v, rows_v, idn_v,
               acc_sp, gs0, gs1):
    sid = lax.axis_index("s")
    pltpu.sync_copy(as_hbm, as_v)
    pltpu.sync_copy(ad_hbm, ad_v)

    rbase = sid * RPT
    cbase = sid * CPT
    pltpu.sync_copy(zp_hbm, acc_sp.at[pl.ds(rbase, RPT)])

    zeros16 = jnp.zeros((16,), F32)

    def zc(i, carry):
        c2d_v[i // 2, pl.ds((i % 2) * 16, 16)] = zeros16
        return carry

    lax.fori_loop(0, 1280, zc, 0)

    plsc.subcore_barrier()

    msv = as_v[pl.ds(NPAD, 16)]
    zero_i = jnp.zeros((16,), jnp.int32)
    npad_i = jnp.full((16,), NPAD, jnp.int32)
    one_f = jnp.full((16,), 1.0, F32)
    fifth_f = jnp.full((16,), 0.2, F32)
    ebase = sid * EW

    def slab_body(sl, cpcn):
        pltpu.sync_copy(src_hbm.at[pl.ds(ebase + sl * SLAB, SLAB)], src_v)
        pltpu.sync_copy(dst_hbm.at[pl.ds(ebase + sl * SLAB, SLAB)], dst_v)

        def egroup(g, cpcn):
            cp, cn = cpcn
            s = src_v[pl.ds(g * 16, 16)]
            d = dst_v[pl.ds(g * 16, 16)]
            a1 = plsc.load_gather(as_v, [s])
            a2 = plsc.load_gather(ad_v, [d])
            pos = (a1 + a2) > 0.0
            off = jnp.where(pos, zero_i, npad_i)
            w = jnp.where(pos, one_f, fifth_f)
            val = jnp.exp(w * (a1 - msv))
            ci = d + off
            plsc.addupdate_scatter(
                c2d_v, [lax.shift_right_logical(ci, 5),
                        lax.bitwise_and(ci, jnp.full((16,), 31, jnp.int32))],
                val)
            gi = s + off
            pc = plsc.all_reduce_population_count(pos)
            kp = lax.squeeze(lax.slice(pc, (0,), (1,)), (0,))
            kn = 16 - kp
            plsc.store_compressed(pg_v.at[pl.ds(cp, 16)], gi, mask=pos)
            plsc.store_compressed(sg_v.at[pl.ds(cp, 16)], d, mask=pos)
            neg = jnp.logical_not(pos)
            plsc.store_compressed(pg_v.at[pl.ds(cn - kn, 16)], gi, mask=neg)
            plsc.store_compressed(sg_v.at[pl.ds(cn - kn, 16)], d, mask=neg)
            return (cp + kp, cn - kn)

        return lax.fori_loop(0, SLAB // 16, egroup, cpcn)

    cp, cn = lax.fori_loop(0, NSLAB, slab_body, (0, TOP))

    dummy_p = jnp.full((16,), N, jnp.int32)
    dummy_n = jnp.full((16,), NPAD + N, jnp.int32)

    def padp(i, carry):
        pg_v[pl.ds(cp + i * 16, 16)] = dummy_p
        sg_v[pl.ds(cp + i * 16, 16)] = dummy_p
        return carry

    def padn(i, carry):
        pg_v[pl.ds(cn - CH + i * 16, 16)] = dummy_n
        sg_v[pl.ds(cn - CH + i * 16, 16)] = dummy_p
        return carry

    lax.fori_loop(0, CH // 16, padp, 0)
    lax.fori_loop(0, CH // 16, padn, 0)

    sixteen = jnp.full((16,), 16, jnp.int32)

    def bld(k, vec):
        idn_v[pl.ds(k * 16, 16)] = vec
        return vec + sixteen

    lax.fori_loop(0, 40, bld, lax.iota(jnp.int32, 16))

    npc = (cp + CH - 1) // CH
    nfs = cn // CH

    def sweep_body(s, carry):
        lo = jnp.where(s < 2, 0, nfs)
        hi = jnp.where(s < 2, npc, TOPC)
        tab = gt_hbm.at[s % 2]

        def start_g(j, buf, gsem):
            pltpu.async_copy(tab.at[pg_v.at[pl.ds(j * CH, CH)]], buf, gsem)

        def wait_g(buf, gsem):
            pltpu.make_async_copy(tab.at[pg_v.at[pl.ds(0, CH)]], buf,
                                  gsem).wait()

        def scat(j, buf):
            pltpu.sync_copy(buf, acc_sp.at[sg_v.at[pl.ds(j * CH, CH)]],
                            add=True)

        @pl.when(hi > lo)
        def _():
            start_g(lo, rows_v.at[0], gs0)

        def body(k, carry2):
            j0 = lo + 2 * k
            j1 = j0 + 1

            @pl.when(j1 < hi)
            def _():
                start_g(j1, rows_v.at[1], gs1)

            wait_g(rows_v.at[0], gs0)
            scat(j0, rows_v.at[0])

            @pl.when(j1 < hi)
            def _():
                @pl.when(j1 + 1 < hi)
                def __():
                    start_g(j1 + 1, rows_v.at[0], gs0)

                wait_g(rows_v.at[1], gs1)
                scat(j1, rows_v.at[1])

            return carry2

        lax.fori_loop(0, (hi - lo + 1) // 2, body, 0)

        plsc.subcore_barrier()
        pltpu.sync_copy(acc_sp.at[pl.ds(rbase, RPT)],
                        p_out.at[s, pl.ds(rbase, RPT)])
        pltpu.sync_copy(zp_hbm, acc_sp.at[pl.ds(rbase, RPT)])
        plsc.subcore_barrier()
        return carry

    lax.fori_loop(0, 4, sweep_body, 0)

    def cchunk(j, carry):
        pltpu.sync_copy(c2d_v.at[pl.ds(j * 128, 128)],
                        acc_sp.at[idn_v.at[pl.ds(j * 128, 128)]], add=True)
        return carry

    lax.fori_loop(0, 5, cchunk, 0)
    plsc.subcore_barrier()

    @pl.when(sid < 8)
    def _():
        pltpu.sync_copy(acc_sp.at[pl.ds(sid * 79, 79)],
                        c_out.at[pl.ds(sid * 79, 79)])


_edge_call = pl.kernel(
    _edge_body,
    out_type=[_sds((4, NPAD, HH), F32), _sds((632, 32), F32)],
    mesh=plsc.VectorSubcoreMesh(core_axis_name="c", subcore_axis_name="s",
                                num_cores=1),
    compiler_params=pltpu.CompilerParams(needs_layout_passes=False,
                                         use_tc_tiling_on_sc=False),
    scratch_types=[
        pltpu.VMEM((SLAB,), jnp.int32),
        pltpu.VMEM((SLAB,), jnp.int32),
        pltpu.VMEM((NPAD + 16,), F32),
        pltpu.VMEM((NPAD,), F32),
        pltpu.VMEM((640, 32), F32),
        pltpu.VMEM((BUF,), jnp.int32),
        pltpu.VMEM((BUF,), jnp.int32),
        pltpu.VMEM((2, CH, HH), F32),
        pltpu.VMEM((640,), jnp.int32),
        pltpu.VMEM_SHARED((NPAD, HH), F32),
        pltpu.SemaphoreType.DMA,
        pltpu.SemaphoreType.DMA,
    ],
)


@jax.jit
def kernel(x, edge_index, batch, W1, a_src1, a_dst1, b1,
           W2, a_src2, a_dst2, b2, Wl, bl):
    loop = jnp.arange(N, dtype=jnp.int32)
    padv = jnp.full((EPAD - E - N,), N, jnp.int32)
    src = jnp.concatenate([edge_index[0].astype(jnp.int32), loop, padv])
    dst = jnp.concatenate([edge_index[1].astype(jnp.int32), loop, padv])
    zp = jnp.zeros((RPT, HH), F32)

    def edge_phase(gt, scal):
        glo = gt[:, :, 0:HH].reshape(N2, HH)
        ghi = gt[:, :, HH:H].reshape(N2, HH)
        gt_all = jnp.stack([glo, ghi])
        as_flat = scal[:, 0]
        ad_flat = scal[0:NPAD, 1]
        p4, c_ = _edge_call(src, dst, as_flat, ad_flat, zp, gt_all)
        c_ = c_.reshape(N2)
        p = jnp.concatenate([p4[0], p4[1], p4[2], p4[3]], axis=1)
        c2 = jnp.stack([c_[0:NPAD], c_[NPAD:N2]], axis=1)
        return p, c2

    gt1, scal1 = _prep_call(x, W1, a_src1.reshape(H, 1), a_dst1.reshape(H, 1))
    p1, c1 = edge_phase(gt1, scal1)
    gt2, scal2 = _mid_call(p1, c1, scal1, b1, W2, a_src2.reshape(H, 1),
                           a_dst2.reshape(H, 1))
    p2, c2 = edge_phase(gt2, scal2)
    return _final_call(p2, c2, scal2, b2,
                       batch.reshape(N, 1).astype(jnp.int32), Wl, bl)
